# Initial kernel scaffold; baseline (speedup 1.0000x reference)
import jax
import jax.numpy as jnp
from jax.experimental import pallas as pl


def kernel(x, edge_index, W1, b1, W2, b2, W3, b3, W4, b4):
    raise NotImplementedError("write your pallas kernel here")


# speedup vs baseline: 3.0509x; 3.0509x over previous
import functools

import jax
import jax.numpy as jnp
from jax import lax
from jax.experimental import pallas as pl
from jax.experimental.pallas import tpu as pltpu
from jax.experimental.pallas import tpu_sc as plsc

N_NODES = 10000
D = 128
NC = 2
NS = 16
NW = NC * NS
K = 128
NP = 10112


def _segment_sum_sc(x, src2d, dst2d, zeros):
    n_rows = src2d.shape[0]
    n_chunks = n_rows // NW
    init_rows = NP // NS

    mesh = plsc.VectorSubcoreMesh(core_axis_name="c", subcore_axis_name="s",
                                  num_cores=NC, num_subcores=NS)

    @functools.partial(
        pl.kernel,
        out_type=jax.ShapeDtypeStruct((NC, NP, D), jnp.float32),
        mesh=mesh,
        scratch_types=[
            pltpu.VMEM((n_chunks, K), jnp.int32),
            pltpu.VMEM((n_chunks, K), jnp.int32),
            pltpu.VMEM((K, D), jnp.float32),
            pltpu.VMEM_SHARED((NP, D), jnp.float32),
            pltpu.SemaphoreType.DMA,
        ],
    )
    def seg_sum(x_hbm, src_hbm, dst_hbm, zeros_hbm, out_hbm,
                sidx, didx, rows, acc, sem):
        c = lax.axis_index("c")
        s = lax.axis_index("s")
        wid = s * NC + c
        pltpu.sync_copy(zeros_hbm.at[pl.ds(s * init_rows, init_rows)],
                        acc.at[pl.ds(s * init_rows, init_rows)])
        pltpu.sync_copy(src_hbm.at[pl.ds(wid * n_chunks, n_chunks)], sidx)
        pltpu.sync_copy(dst_hbm.at[pl.ds(wid * n_chunks, n_chunks)], didx)
        plsc.subcore_barrier()

        @pl.loop(0, n_chunks)
        def _(j):
            pltpu.async_copy(x_hbm.at[sidx.at[j]], rows, sem).wait()
            pltpu.sync_copy(rows, acc.at[didx.at[j]], add=True)

        plsc.subcore_barrier()
        pltpu.sync_copy(acc.at[pl.ds(s * init_rows, init_rows)],
                        out_hbm.at[c, pl.ds(s * init_rows, init_rows)])

    return seg_sum(x, src2d, dst2d, zeros)


def _mlp_body(final, x_ref, a0_ref, a1_ref, wa_ref, ba_ref, wb_ref, bb_ref,
              o_ref):
    h = x_ref[...] + a0_ref[...] + a1_ref[...]
    t = jnp.dot(h, wa_ref[...], preferred_element_type=jnp.float32)
    t = jnp.maximum(t + ba_ref[...], 0.0)
    z = jnp.dot(t, wb_ref[...], preferred_element_type=jnp.float32)
    z = z + bb_ref[...]
    if final:
        m = jnp.max(z, axis=1, keepdims=True)
        z = z - m
        z = z - jnp.log(jnp.sum(jnp.exp(z), axis=1, keepdims=True))
    else:
        z = jnp.maximum(z, 0.0)
    o_ref[...] = z


def _mlp_tc(x, a0, a1, Wa, ba, Wb, bb, final):
    blk = 1000
    grid = (N_NODES // blk,)
    row_spec = pl.BlockSpec((blk, D), lambda i: (i, 0))
    full_spec = pl.BlockSpec((D, D), lambda i: (0, 0))
    bias_spec = pl.BlockSpec((1, D), lambda i: (0, 0))
    return pl.pallas_call(
        functools.partial(_mlp_body, final),
        grid=grid,
        in_specs=[row_spec, row_spec, row_spec,
                  full_spec, bias_spec, full_spec, bias_spec],
        out_specs=row_spec,
        out_shape=jax.ShapeDtypeStruct((N_NODES, D), jnp.float32),
    )(x, a0, a1, Wa, ba.reshape(1, D), Wb, bb.reshape(1, D))


def kernel(x, edge_index, W1, b1, W2, b2, W3, b3, W4, b4):
    src = edge_index[0].astype(jnp.int32)
    dst = edge_index[1].astype(jnp.int32)
    e = src.shape[0]
    unit = NW * K * 8
    e_pad = ((e + unit - 1) // unit) * unit
    src2d = jnp.pad(src, (0, e_pad - e)).reshape(e_pad // K, K)
    dst2d = jnp.pad(dst, (0, e_pad - e),
                    constant_values=N_NODES).reshape(e_pad // K, K)
    zeros = jnp.zeros((NP, D), jnp.float32)

    p1 = _segment_sum_sc(x, src2d, dst2d, zeros)
    h = _mlp_tc(x, p1[0], p1[1], W1, b1, W2, b2, final=False)
    p2 = _segment_sum_sc(h, src2d, dst2d, zeros)
    return _mlp_tc(h, p2[0], p2[1], W3, b3, W4, b4, final=True)
